# Initial kernel scaffold; baseline (speedup 1.0000x reference)
import jax
import jax.numpy as jnp
from jax.experimental import pallas as pl


def kernel(index, hidden_states, router_w, ws):
    raise NotImplementedError("write your pallas kernel here")


# speedup vs baseline: 1.1497x; 1.1497x over previous
import jax
import jax.numpy as jnp
from jax.experimental import pallas as pl
from jax.experimental.pallas import tpu as pltpu

E = 8
H = 1024
I = 1024
T = 32


def _dot_t(a, b):
    return jax.lax.dot_general(
        a, b, (((1,), (1,)), ((), ())), preferred_element_type=jnp.float32
    )


def _moe_kernel(x_ref, rw_ref, w_ref, out_ref, fw_ref, h_ref):
    e = pl.program_id(0)
    phase = pl.program_id(1)

    @pl.when(jnp.logical_and(e == 0, phase == 0))
    def _router():
        logits = _dot_t(x_ref[...], rw_ref[...])
        m = jnp.max(logits, axis=-1, keepdims=True)
        p = jnp.exp(logits - m)
        s = p / jnp.sum(p, axis=-1, keepdims=True)
        lane = jax.lax.broadcasted_iota(jnp.int32, (T, E), 1)
        m1 = jnp.max(s, axis=-1, keepdims=True)
        i1 = jnp.min(jnp.where(s == m1, lane, E), axis=-1, keepdims=True)
        s_rest = jnp.where(lane == i1, -1.0, s)
        m2 = jnp.max(s_rest, axis=-1, keepdims=True)
        i2 = jnp.min(jnp.where(s_rest == m2, lane, E), axis=-1, keepdims=True)
        fw = jnp.where(lane == i1, m1, 0.0) + jnp.where(lane == i2, m2, 0.0)
        fw_ref[...] = fw / (m1 + m2)

    blk = w_ref[0]

    @pl.when(phase == 0)
    def _p0():
        h_ref[...] = _dot_t(x_ref[...], blk)

    @pl.when(phase == 1)
    def _p1():
        h1 = h_ref[...]
        h2 = _dot_t(x_ref[...], blk)
        h_ref[...] = (h1 * jax.lax.logistic(h1)) * h2

    @pl.when(phase == 2)
    def _p2():
        part = _dot_t(h_ref[...], blk)
        lane = jax.lax.broadcasted_iota(jnp.int32, (T, E), 1)
        wcol = jnp.sum(
            jnp.where(lane == e, fw_ref[...], 0.0), axis=-1, keepdims=True
        )
        contrib = part * wcol

        @pl.when(e == 0)
        def _init():
            out_ref[...] = contrib

        @pl.when(e != 0)
        def _acc():
            out_ref[...] = out_ref[...] + contrib


def kernel(index, hidden_states, router_w, ws):
    del index
    x = hidden_states.reshape(T, H)
    rw = router_w.reshape(E, H)
    w = ws.reshape(E, 3 * I, H)
    return pl.pallas_call(
        _moe_kernel,
        grid=(E, 3),
        in_specs=[
            pl.BlockSpec((T, H), lambda e, p: (0, 0)),
            pl.BlockSpec((E, H), lambda e, p: (0, 0)),
            pl.BlockSpec((1, I, H), lambda e, p: (e, p, 0)),
        ],
        out_specs=pl.BlockSpec((T, H), lambda e, p: (0, 0)),
        out_shape=jax.ShapeDtypeStruct((T, H), jnp.float32),
        scratch_shapes=[
            pltpu.VMEM((T, E), jnp.float32),
            pltpu.VMEM((T, I), jnp.float32),
        ],
    )(x, rw, w)


# speedup vs baseline: 1.1517x; 1.0017x over previous
import jax
import jax.numpy as jnp
from jax.experimental import pallas as pl
from jax.experimental.pallas import tpu as pltpu

E = 8
H = 1024
I = 1024
T = 32


def _dot_t(a, b):
    return jax.lax.dot_general(
        a.astype(jnp.bfloat16),
        b.astype(jnp.bfloat16),
        (((1,), (1,)), ((), ())),
        preferred_element_type=jnp.float32,
    )


def _moe_kernel(x_ref, rw_ref, w_ref, out_ref, fw_ref, h_ref):
    e = pl.program_id(0)
    phase = pl.program_id(1)

    @pl.when(jnp.logical_and(e == 0, phase == 0))
    def _router():
        logits = _dot_t(x_ref[...], rw_ref[...])
        m = jnp.max(logits, axis=-1, keepdims=True)
        p = jnp.exp(logits - m)
        s = p / jnp.sum(p, axis=-1, keepdims=True)
        lane = jax.lax.broadcasted_iota(jnp.int32, (T, E), 1)
        m1 = jnp.max(s, axis=-1, keepdims=True)
        i1 = jnp.min(jnp.where(s == m1, lane, E), axis=-1, keepdims=True)
        s_rest = jnp.where(lane == i1, -1.0, s)
        m2 = jnp.max(s_rest, axis=-1, keepdims=True)
        i2 = jnp.min(jnp.where(s_rest == m2, lane, E), axis=-1, keepdims=True)
        fw = jnp.where(lane == i1, m1, 0.0) + jnp.where(lane == i2, m2, 0.0)
        fw_ref[...] = fw / (m1 + m2)

    blk = w_ref[0]

    @pl.when(phase == 0)
    def _p0():
        h_ref[...] = _dot_t(x_ref[...], blk)

    @pl.when(phase == 1)
    def _p1():
        h1 = h_ref[...]
        h2 = _dot_t(x_ref[...], blk)
        h_ref[...] = (h1 * jax.lax.logistic(h1)) * h2

    @pl.when(phase == 2)
    def _p2():
        part = _dot_t(h_ref[...], blk)
        lane = jax.lax.broadcasted_iota(jnp.int32, (T, E), 1)
        wcol = jnp.sum(
            jnp.where(lane == e, fw_ref[...], 0.0), axis=-1, keepdims=True
        )
        contrib = part * wcol

        @pl.when(e == 0)
        def _init():
            out_ref[...] = contrib

        @pl.when(e != 0)
        def _acc():
            out_ref[...] = out_ref[...] + contrib


def kernel(index, hidden_states, router_w, ws):
    del index
    x = hidden_states.reshape(T, H)
    rw = router_w.reshape(E, H)
    w = ws.reshape(E, 3 * I, H)
    return pl.pallas_call(
        _moe_kernel,
        grid=(E, 3),
        in_specs=[
            pl.BlockSpec((T, H), lambda e, p: (0, 0)),
            pl.BlockSpec((E, H), lambda e, p: (0, 0)),
            pl.BlockSpec((1, I, H), lambda e, p: (e, p, 0)),
        ],
        out_specs=pl.BlockSpec((T, H), lambda e, p: (0, 0)),
        out_shape=jax.ShapeDtypeStruct((T, H), jnp.float32),
        scratch_shapes=[
            pltpu.VMEM((T, E), jnp.float32),
            pltpu.VMEM((T, I), jnp.float32),
        ],
    )(x, rw, w)


# speedup vs baseline: 1.4641x; 1.2712x over previous
import jax
import jax.numpy as jnp
from jax.experimental import pallas as pl
from jax.experimental.pallas import tpu as pltpu

E = 8
H = 1024
I = 1024
T = 32
R = 128
CH = R * H
N1 = I // R


def _bf(a):
    return a.astype(jnp.bfloat16)


def _stack(v):
    return jnp.concatenate([v[:, r * H : (r + 1) * H] for r in range(R)], axis=0)


def _moe_kernel(x_ref, rw_ref, ws_ref, w2j_ref, out_ref, h1t_ref, a2_ref, fwb_ref):
    j = pl.program_id(0)

    @pl.when(j == 0)
    def _router():
        logits = jax.lax.dot_general(
            x_ref[...], rw_ref[...], (((1,), (1,)), ((), ())),
            preferred_element_type=jnp.float32,
            precision=jax.lax.Precision.HIGHEST,
        )
        m = jnp.max(logits, axis=-1, keepdims=True)
        p = jnp.exp(logits - m)
        s = p / jnp.sum(p, axis=-1, keepdims=True)
        lane = jax.lax.broadcasted_iota(jnp.int32, (T, E), 1)
        m1 = jnp.max(s, axis=-1, keepdims=True)
        i1 = jnp.min(jnp.where(s == m1, lane, E), axis=-1, keepdims=True)
        s_rest = jnp.where(lane == i1, -1.0, s)
        m2 = jnp.max(s_rest, axis=-1, keepdims=True)
        i2 = jnp.min(jnp.where(s_rest == m2, lane, E), axis=-1, keepdims=True)
        fw = jnp.where(lane == i1, m1, 0.0) + jnp.where(lane == i2, m2, 0.0)
        fw = fw / (m1 + m2)
        fwt = fw.T
        fwb_ref[...] = jnp.concatenate([fwt for _ in range(R)], axis=0)

    @pl.when(j < N1)
    def _w1():
        w_stack = _stack(ws_ref[...])
        h1t = jax.lax.dot_general(
            _bf(w_stack), _bf(x_ref[...]), (((1,), (1,)), ((), ())),
            preferred_element_type=jnp.float32,
        )
        h1t_ref[pl.ds(j * 8 * R, 8 * R), :] = h1t

    @pl.when(jnp.logical_and(j >= N1, j < 2 * N1))
    def _v1():
        jj = j - N1
        w_stack = _stack(ws_ref[...])
        h2t = jax.lax.dot_general(
            _bf(w_stack), _bf(x_ref[...]), (((1,), (1,)), ((), ())),
            preferred_element_type=jnp.float32,
        )
        h1t = h1t_ref[pl.ds(jj * 8 * R, 8 * R), :]
        at = (h1t * jax.lax.logistic(h1t)) * h2t * fwb_ref[...]
        a2_ref[:, pl.ds(jj * 8 * R, 8 * R)] = at.T

    @pl.when(j >= 2 * N1)
    def _w2():
        jj = j - 2 * N1
        part = jax.lax.dot_general(
            _bf(a2_ref[:, pl.ds(jj * 8 * R, 8 * R)]), _bf(w2j_ref[...]),
            (((1,), (0,)), ((), ())),
            preferred_element_type=jnp.float32,
        )

        @pl.when(jj == 0)
        def _init():
            out_ref[...] = part

        @pl.when(jj != 0)
        def _acc():
            out_ref[...] = out_ref[...] + part


def kernel(index, hidden_states, router_w, ws):
    del index
    x = hidden_states.reshape(T, H)
    rw = router_w.reshape(E, H)
    ws2d = ws.reshape(E, 3 * I * H)
    w2j = ws2d[:, 2 * I * H :].reshape(E, H, I).transpose(2, 0, 1).reshape(I * E, H)
    return pl.pallas_call(
        _moe_kernel,
        grid=(3 * N1,),
        in_specs=[
            pl.BlockSpec((T, H), lambda j: (0, 0)),
            pl.BlockSpec((E, H), lambda j: (0, 0)),
            pl.BlockSpec((E, CH), lambda j: (0, jnp.minimum(j, 2 * N1 - 1))),
            pl.BlockSpec(
                (8 * R, H), lambda j: (jnp.maximum(j - 2 * N1, 0), 0)
            ),
        ],
        out_specs=pl.BlockSpec((T, H), lambda j: (0, 0)),
        out_shape=jax.ShapeDtypeStruct((T, H), jnp.float32),
        scratch_shapes=[
            pltpu.VMEM((E * I, T), jnp.float32),
            pltpu.VMEM((T, E * I), jnp.float32),
            pltpu.VMEM((E * R, T), jnp.float32),
        ],
    )(x, rw, ws2d, w2j)


# speedup vs baseline: 1.9275x; 1.3165x over previous
import jax
import jax.numpy as jnp
from jax.experimental import pallas as pl
from jax.experimental.pallas import tpu as pltpu

E = 8
H = 1024
I = 1024
T = 32
R = 256
CH = R * H
N1 = I // R


def _bf(a):
    return a.astype(jnp.bfloat16)


def _stack(v):
    return jnp.concatenate([v[:, r * H : (r + 1) * H] for r in range(R)], axis=0)


def _moe_kernel(x_ref, rw_ref, ws_ref, w2j_ref, out_ref, h1t_ref, a2_ref, fwb_ref):
    j = pl.program_id(0)

    @pl.when(j == 0)
    def _router():
        logits = jax.lax.dot_general(
            x_ref[...], rw_ref[...], (((1,), (1,)), ((), ())),
            preferred_element_type=jnp.float32,
            precision=jax.lax.Precision.HIGHEST,
        )
        m = jnp.max(logits, axis=-1, keepdims=True)
        p = jnp.exp(logits - m)
        s = p / jnp.sum(p, axis=-1, keepdims=True)
        lane = jax.lax.broadcasted_iota(jnp.int32, (T, E), 1)
        m1 = jnp.max(s, axis=-1, keepdims=True)
        i1 = jnp.min(jnp.where(s == m1, lane, E), axis=-1, keepdims=True)
        s_rest = jnp.where(lane == i1, -1.0, s)
        m2 = jnp.max(s_rest, axis=-1, keepdims=True)
        i2 = jnp.min(jnp.where(s_rest == m2, lane, E), axis=-1, keepdims=True)
        fw = jnp.where(lane == i1, m1, 0.0) + jnp.where(lane == i2, m2, 0.0)
        fw = fw / (m1 + m2)
        fwt = fw.T
        fwb_ref[...] = jnp.concatenate([fwt for _ in range(R)], axis=0)

    @pl.when(j < N1)
    def _w1():
        w_stack = _stack(ws_ref[...])
        h1t = jax.lax.dot_general(
            _bf(w_stack), _bf(x_ref[...]), (((1,), (1,)), ((), ())),
            preferred_element_type=jnp.float32,
        )
        h1t_ref[pl.ds(j * 8 * R, 8 * R), :] = h1t

    @pl.when(jnp.logical_and(j >= N1, j < 2 * N1))
    def _v1():
        jj = j - N1
        w_stack = _stack(ws_ref[...])
        h2t = jax.lax.dot_general(
            _bf(w_stack), _bf(x_ref[...]), (((1,), (1,)), ((), ())),
            preferred_element_type=jnp.float32,
        )
        h1t = h1t_ref[pl.ds(jj * 8 * R, 8 * R), :]
        at = (h1t * jax.lax.logistic(h1t)) * h2t * fwb_ref[...]
        a2_ref[:, pl.ds(jj * 8 * R, 8 * R)] = at.T

    @pl.when(j >= 2 * N1)
    def _w2():
        jj = j - 2 * N1
        part = jax.lax.dot_general(
            _bf(a2_ref[:, pl.ds(jj * 8 * R, 8 * R)]), w2j_ref[...],
            (((1,), (0,)), ((), ())),
            preferred_element_type=jnp.float32,
        )

        @pl.when(jj == 0)
        def _init():
            out_ref[...] = part

        @pl.when(jj != 0)
        def _acc():
            out_ref[...] = out_ref[...] + part


def kernel(index, hidden_states, router_w, ws):
    del index
    x = hidden_states.reshape(T, H)
    rw = router_w.reshape(E, H)
    ws2d = ws.reshape(E, 3 * I * H)
    w2j = (
        ws2d[:, 2 * I * H :]
        .astype(jnp.bfloat16)
        .reshape(E, H, I)
        .transpose(2, 0, 1)
        .reshape(I * E, H)
    )
    return pl.pallas_call(
        _moe_kernel,
        grid=(3 * N1,),
        in_specs=[
            pl.BlockSpec((T, H), lambda j: (0, 0)),
            pl.BlockSpec((E, H), lambda j: (0, 0)),
            pl.BlockSpec((E, CH), lambda j: (0, jnp.minimum(j, 2 * N1 - 1))),
            pl.BlockSpec(
                (8 * R, H), lambda j: (jnp.maximum(j - 2 * N1, 0), 0)
            ),
        ],
        out_specs=pl.BlockSpec((T, H), lambda j: (0, 0)),
        out_shape=jax.ShapeDtypeStruct((T, H), jnp.float32),
        scratch_shapes=[
            pltpu.VMEM((E * I, T), jnp.float32),
            pltpu.VMEM((T, E * I), jnp.float32),
            pltpu.VMEM((E * R, T), jnp.float32),
        ],
    )(x, rw, ws2d, w2j)


# speedup vs baseline: 2.0393x; 1.0580x over previous
import jax
import jax.numpy as jnp
from jax.experimental import pallas as pl
from jax.experimental.pallas import tpu as pltpu

E = 8
H = 1024
I = 1024
T = 32
R = 256
CH = R * H
N1 = I // R


def _bf(a):
    return a.astype(jnp.bfloat16)


def _stack(v):
    return jnp.concatenate([v[:, r * H : (r + 1) * H] for r in range(R)], axis=0)


def _a2_kernel(x_ref, rw_ref, ws_ref, a2_ref, h1t_ref, fwb_ref):
    j = pl.program_id(0)

    @pl.when(j == 0)
    def _router():
        logits = jax.lax.dot_general(
            x_ref[...], rw_ref[...], (((1,), (1,)), ((), ())),
            preferred_element_type=jnp.float32,
            precision=jax.lax.Precision.HIGHEST,
        )
        m = jnp.max(logits, axis=-1, keepdims=True)
        p = jnp.exp(logits - m)
        s = p / jnp.sum(p, axis=-1, keepdims=True)
        lane = jax.lax.broadcasted_iota(jnp.int32, (T, E), 1)
        m1 = jnp.max(s, axis=-1, keepdims=True)
        i1 = jnp.min(jnp.where(s == m1, lane, E), axis=-1, keepdims=True)
        s_rest = jnp.where(lane == i1, -1.0, s)
        m2 = jnp.max(s_rest, axis=-1, keepdims=True)
        i2 = jnp.min(jnp.where(s_rest == m2, lane, E), axis=-1, keepdims=True)
        fw = jnp.where(lane == i1, m1, 0.0) + jnp.where(lane == i2, m2, 0.0)
        fw = fw / (m1 + m2)
        fwt = fw.T
        fwb_ref[...] = jnp.concatenate([fwt for _ in range(R)], axis=0)

    @pl.when(j < N1)
    def _w1():
        w_stack = _stack(ws_ref[...])
        h1t = jax.lax.dot_general(
            _bf(w_stack), _bf(x_ref[...]), (((1,), (1,)), ((), ())),
            preferred_element_type=jnp.float32,
        )
        h1t_ref[pl.ds(j * 8 * R, 8 * R), :] = h1t

    @pl.when(j >= N1)
    def _v1():
        jj = j - N1
        w_stack = _stack(ws_ref[...])
        h2t = jax.lax.dot_general(
            _bf(w_stack), _bf(x_ref[...]), (((1,), (1,)), ((), ())),
            preferred_element_type=jnp.float32,
        )
        h1t = h1t_ref[pl.ds(jj * 8 * R, 8 * R), :]
        at = (h1t * jax.lax.logistic(h1t)) * h2t * fwb_ref[...]
        a2_ref[:, pl.ds(jj * 8 * R, 8 * R)] = _bf(at.T)


def _out_kernel(a2_ref, w2j_ref, out_ref):
    j = pl.program_id(0)
    part = jax.lax.dot_general(
        a2_ref[:, pl.ds(j * 8 * R, 8 * R)], w2j_ref[...],
        (((1,), (0,)), ((), ())),
        preferred_element_type=jnp.float32,
    )

    @pl.when(j == 0)
    def _init():
        out_ref[...] = part

    @pl.when(j != 0)
    def _acc():
        out_ref[...] = out_ref[...] + part


def kernel(index, hidden_states, router_w, ws):
    del index
    x = hidden_states.reshape(T, H)
    rw = router_w.reshape(E, H)
    ws2d = ws.reshape(E, 3 * I * H)
    w2j = (
        ws2d[:, 2 * I * H :]
        .astype(jnp.bfloat16)
        .reshape(E, H, I)
        .transpose(2, 0, 1)
        .reshape(I * E, H)
    )
    a2 = pl.pallas_call(
        _a2_kernel,
        grid=(2 * N1,),
        in_specs=[
            pl.BlockSpec((T, H), lambda j: (0, 0)),
            pl.BlockSpec((E, H), lambda j: (0, 0)),
            pl.BlockSpec((E, CH), lambda j: (0, j)),
        ],
        out_specs=pl.BlockSpec((T, E * I), lambda j: (0, 0)),
        out_shape=jax.ShapeDtypeStruct((T, E * I), jnp.bfloat16),
        scratch_shapes=[
            pltpu.VMEM((E * I, T), jnp.float32),
            pltpu.VMEM((E * R, T), jnp.float32),
        ],
    )(x, rw, ws2d)
    return pl.pallas_call(
        _out_kernel,
        grid=(N1,),
        in_specs=[
            pl.BlockSpec((T, E * I), lambda j: (0, 0)),
            pl.BlockSpec((8 * R, H), lambda j: (j, 0)),
        ],
        out_specs=pl.BlockSpec((T, H), lambda j: (0, 0)),
        out_shape=jax.ShapeDtypeStruct((T, H), jnp.float32),
    )(a2, w2j)
